# baseline (speedup 1.0000x reference)
import functools

import jax
import jax.numpy as jnp
from jax import lax
from jax.experimental import pallas as pl
from jax.experimental.pallas import tpu as pltpu
from jax.experimental.pallas import tpu_sc as plsc

N_TOKENS = 32768
D = 768
E = 64
K = 8
L = 16
NC = 2
NS = 16
NW = NC * NS
CPW = N_TOKENS // NW
NCHUNK = CPW // L

TC_TILE = 4096
N_SLABS = 4


def _probs_body(x_ref, w_ref, b_ref, out_ref):
    s = lax.dot_general(w_ref[...], x_ref[...],
                        (((1,), (1,)), ((), ())),
                        preferred_element_type=jnp.float32)
    s = s + b_ref[...]
    m = jnp.max(s, axis=0, keepdims=True)
    z = jnp.exp(s - m)
    out_ref[...] = z / jnp.sum(z, axis=0, keepdims=True)


def _probs_tc(x, w, b2, slab, slab_tokens):
    blk0 = slab * (slab_tokens // TC_TILE)
    return pl.pallas_call(
        _probs_body,
        grid=(slab_tokens // TC_TILE,),
        in_specs=[
            pl.BlockSpec((TC_TILE, D), lambda i: (blk0 + i, 0)),
            pl.BlockSpec((E, D), lambda i: (0, 0)),
            pl.BlockSpec((E, 1), lambda i: (0, 0)),
        ],
        out_specs=pl.BlockSpec((E, TC_TILE), lambda i: (0, i)),
        out_shape=jax.ShapeDtypeStruct((E, slab_tokens), jnp.float32),
    )(x, w, b2)


_SORT8 = [(0, 1), (2, 3), (4, 5), (6, 7),
          (0, 2), (1, 3), (4, 6), (5, 7),
          (1, 2), (5, 6),
          (0, 4), (1, 5), (2, 6), (3, 7),
          (2, 4), (3, 5),
          (1, 2), (3, 4), (5, 6)]
_BITONIC8 = [(0, 4), (1, 5), (2, 6), (3, 7),
             (0, 2), (1, 3), (4, 6), (5, 7),
             (0, 1), (2, 3), (4, 5), (6, 7)]


def _cs_desc(v, i, j):
    hi = jnp.maximum(v[i], v[j])
    lo = jnp.minimum(v[i], v[j])
    v[i] = hi
    v[j] = lo


def _sort8_desc(v):
    for i, j in _SORT8:
        _cs_desc(v, i, j)
    return v


def _merge_top8(a, b):
    out = [jnp.maximum(a[i], b[7 - i]) for i in range(8)]
    for i, j in _BITONIC8:
        _cs_desc(out, i, j)
    return out


def _top8_of_64(vals):
    groups = [_sort8_desc([vals[g * 8 + i] for i in range(8)])
              for g in range(8)]
    r = [_merge_top8(groups[2 * i], groups[2 * i + 1]) for i in range(4)]
    r = [_merge_top8(r[0], r[1]), _merge_top8(r[2], r[3])]
    return _merge_top8(r[0], r[1])


def _topk_sc_body(probs_hbm, pout_hbm, iout_hbm, pv, pov, iov, *, cpw):
    wid = lax.axis_index("s") * NC + lax.axis_index("c")
    base = wid * cpw
    pltpu.sync_copy(probs_hbm.at[:, pl.ds(base, cpw)], pv)

    def chunk(i, carry):
        off = i * L
        rows = off + lax.iota(jnp.int32, L)
        packed = []
        for e in range(E):
            bits = plsc.bitcast(pv[e, pl.ds(off, L)], jnp.int32)
            packed.append((bits & jnp.int32(-64)) | jnp.int32(63 - e))
        top = _top8_of_64(packed)
        eidx = []
        ps = []
        for j in range(K):
            ej = jnp.int32(63) - (top[j] & jnp.int32(63))
            eidx.append(ej)
            ps.append(plsc.load_gather(pv, [ej, rows]))
        qs = [jnp.exp(p - ps[0]) for p in ps]
        z = qs[0]
        for j in range(1, K):
            z = z + qs[j]
        for j in range(K):
            cols = jnp.full((L,), j, jnp.int32)
            plsc.store_scatter(pov, [rows, cols], qs[j] / z)
            plsc.store_scatter(iov, [rows, cols], eidx[j])
        return carry

    lax.fori_loop(0, cpw // L, chunk, 0)
    pltpu.sync_copy(pov, pout_hbm.at[pl.ds(base, cpw), :])
    pltpu.sync_copy(iov, iout_hbm.at[pl.ds(base, cpw), :])


@functools.cache
def _topk_sc(n_tokens):
    cpw = n_tokens // NW
    return pl.kernel(
        functools.partial(_topk_sc_body, cpw=cpw),
        out_type=[jax.ShapeDtypeStruct((n_tokens, K), jnp.float32),
                  jax.ShapeDtypeStruct((n_tokens, K), jnp.int32)],
        mesh=plsc.VectorSubcoreMesh(core_axis_name="c", subcore_axis_name="s",
                                    num_cores=NC, num_subcores=NS),
        scratch_types=[pltpu.VMEM((E, cpw), jnp.float32),
                       pltpu.VMEM((cpw, K), jnp.float32),
                       pltpu.VMEM((cpw, K), jnp.int32)],
        compiler_params=pltpu.CompilerParams(needs_layout_passes=False,
                                             use_tc_tiling_on_sc=False),
    )


def kernel(x_local, W, b):
    x = x_local.reshape(x_local.shape[0], -1)
    b2 = b.reshape(E, 1)
    slab_tokens = N_TOKENS // N_SLABS
    pouts, iouts = [], []
    for s in range(N_SLABS):
        probs_t = _probs_tc(x, W, b2, s, slab_tokens)
        po, io = _topk_sc(slab_tokens)(probs_t)
        pouts.append(po)
        iouts.append(io)
    if N_SLABS == 1:
        return pouts[0], iouts[0]
    return jnp.concatenate(pouts, axis=0), jnp.concatenate(iouts, axis=0)


# speedup vs baseline: 1.0207x; 1.0207x over previous
import functools

import jax
import jax.numpy as jnp
from jax import lax
from jax.experimental import pallas as pl
from jax.experimental.pallas import tpu as pltpu
from jax.experimental.pallas import tpu_sc as plsc

N_TOKENS = 32768
D = 768
E = 64
K = 8
L = 16
NC = 2
NS = 16
NW = NC * NS
CPW = N_TOKENS // NW
NCHUNK = CPW // L

TC_TILE = 4096
N_SLABS = 2


def _probs_body(x_ref, w_ref, b_ref, out_ref):
    s = lax.dot_general(w_ref[...], x_ref[...],
                        (((1,), (1,)), ((), ())),
                        preferred_element_type=jnp.float32)
    s = s + b_ref[...]
    m = jnp.max(s, axis=0, keepdims=True)
    z = jnp.exp(s - m)
    out_ref[...] = z / jnp.sum(z, axis=0, keepdims=True)


def _probs_tc(x, w, b2, slab, slab_tokens):
    blk0 = slab * (slab_tokens // TC_TILE)
    return pl.pallas_call(
        _probs_body,
        grid=(slab_tokens // TC_TILE,),
        in_specs=[
            pl.BlockSpec((TC_TILE, D), lambda i: (blk0 + i, 0)),
            pl.BlockSpec((E, D), lambda i: (0, 0)),
            pl.BlockSpec((E, 1), lambda i: (0, 0)),
        ],
        out_specs=pl.BlockSpec((E, TC_TILE), lambda i: (0, i)),
        out_shape=jax.ShapeDtypeStruct((E, slab_tokens), jnp.float32),
    )(x, w, b2)


_SORT8 = [(0, 1), (2, 3), (4, 5), (6, 7),
          (0, 2), (1, 3), (4, 6), (5, 7),
          (1, 2), (5, 6),
          (0, 4), (1, 5), (2, 6), (3, 7),
          (2, 4), (3, 5),
          (1, 2), (3, 4), (5, 6)]
_BITONIC8 = [(0, 4), (1, 5), (2, 6), (3, 7),
             (0, 2), (1, 3), (4, 6), (5, 7),
             (0, 1), (2, 3), (4, 5), (6, 7)]


def _cs_desc(v, i, j):
    hi = jnp.maximum(v[i], v[j])
    lo = jnp.minimum(v[i], v[j])
    v[i] = hi
    v[j] = lo


def _sort8_desc(v):
    for i, j in _SORT8:
        _cs_desc(v, i, j)
    return v


def _merge_top8(a, b):
    out = [jnp.maximum(a[i], b[7 - i]) for i in range(8)]
    for i, j in _BITONIC8:
        _cs_desc(out, i, j)
    return out


def _top8_of_64(vals):
    groups = [_sort8_desc([vals[g * 8 + i] for i in range(8)])
              for g in range(8)]
    r = [_merge_top8(groups[2 * i], groups[2 * i + 1]) for i in range(4)]
    r = [_merge_top8(r[0], r[1]), _merge_top8(r[2], r[3])]
    return _merge_top8(r[0], r[1])


def _topk_sc_body(probs_hbm, pout_hbm, iout_hbm, pv, pov, iov, *, cpw):
    wid = lax.axis_index("s") * NC + lax.axis_index("c")
    base = wid * cpw
    pltpu.sync_copy(probs_hbm.at[:, pl.ds(base, cpw)], pv)

    def chunk(i, carry):
        off = i * L
        rows = off + lax.iota(jnp.int32, L)
        packed = []
        for e in range(E):
            bits = plsc.bitcast(pv[e, pl.ds(off, L)], jnp.int32)
            u = (bits & jnp.int32(-64)) | jnp.int32(63 - e)
            packed.append(plsc.bitcast(u, jnp.float32))
        top = _top8_of_64(packed)
        eidx = []
        ps = []
        for j in range(K):
            ej = jnp.int32(63) - (plsc.bitcast(top[j], jnp.int32)
                                  & jnp.int32(63))
            eidx.append(ej)
            ps.append(plsc.load_gather(pv, [ej, rows]))
        qs = [jnp.exp(p - ps[0]) for p in ps]
        z = qs[0]
        for j in range(1, K):
            z = z + qs[j]
        for j in range(K):
            cols = jnp.full((L,), j, jnp.int32)
            plsc.store_scatter(pov, [rows, cols], qs[j] / z)
            plsc.store_scatter(iov, [rows, cols], eidx[j])
        return carry

    lax.fori_loop(0, cpw // L, chunk, 0)
    pltpu.sync_copy(pov, pout_hbm.at[pl.ds(base, cpw), :])
    pltpu.sync_copy(iov, iout_hbm.at[pl.ds(base, cpw), :])


@functools.cache
def _topk_sc(n_tokens):
    cpw = n_tokens // NW
    return pl.kernel(
        functools.partial(_topk_sc_body, cpw=cpw),
        out_type=[jax.ShapeDtypeStruct((n_tokens, K), jnp.float32),
                  jax.ShapeDtypeStruct((n_tokens, K), jnp.int32)],
        mesh=plsc.VectorSubcoreMesh(core_axis_name="c", subcore_axis_name="s",
                                    num_cores=NC, num_subcores=NS),
        scratch_types=[pltpu.VMEM((E, cpw), jnp.float32),
                       pltpu.VMEM((cpw, K), jnp.float32),
                       pltpu.VMEM((cpw, K), jnp.int32)],
        compiler_params=pltpu.CompilerParams(needs_layout_passes=False,
                                             use_tc_tiling_on_sc=False),
    )


def kernel(x_local, W, b):
    x = x_local.reshape(x_local.shape[0], -1)
    b2 = b.reshape(E, 1)
    slab_tokens = N_TOKENS // N_SLABS
    pouts, iouts = [], []
    for s in range(N_SLABS):
        probs_t = _probs_tc(x, W, b2, s, slab_tokens)
        po, io = _topk_sc(slab_tokens)(probs_t)
        pouts.append(po)
        iouts.append(io)
    if N_SLABS == 1:
        return pouts[0], iouts[0]
    return jnp.concatenate(pouts, axis=0), jnp.concatenate(iouts, axis=0)
